# baseline (speedup 1.0000x reference)
import functools

import jax
import jax.numpy as jnp
from jax import lax
from jax.experimental import pallas as pl
from jax.experimental.pallas import tpu as pltpu
from jax.experimental.pallas import tpu_sc as plsc

BATCH = 16384
HIST = 50
DIM = 64

NC = 2
NS = 16
NW = NC * NS

B_PER_W = BATCH // NW
IDX_PER_W = B_PER_W * HIST
BT_PER_W = B_PER_W // 128
NSTEP = HIST * BT_PER_W


def _build_sc_call():
    mesh = plsc.VectorSubcoreMesh(core_axis_name="c", subcore_axis_name="s")

    @functools.partial(
        pl.kernel,
        mesh=mesh,
        compiler_params=pltpu.CompilerParams(
            needs_layout_passes=False, use_tc_tiling_on_sc=False
        ),
        out_type=jax.ShapeDtypeStruct(
            (HIST, DIM // 8, BATCH // 128, 8, 128), jnp.float32
        ),
        scratch_types=[
            pltpu.VMEM((IDX_PER_W,), jnp.int32),
            pltpu.VMEM((2, 128), jnp.int32),
            pltpu.VMEM((4, 128, DIM), jnp.float32),
            pltpu.VMEM((2, BT_PER_W, DIM, 129), jnp.float32),
            pltpu.SemaphoreType.DMA,
            pltpu.SemaphoreType.DMA,
            pltpu.SemaphoreType.DMA,
        ],
    )
    def emb_kernel(
        x_hbm, table_hbm, out_hbm, idx_v, glist_v, rows_v, blk_v,
        gsem, osem0, osem1,
    ):
        wid = lax.axis_index("s") * NC + lax.axis_index("c")
        pltpu.sync_copy(x_hbm.at[pl.ds(wid * IDX_PER_W, IDX_PER_W)], idx_v)

        lane16 = lax.iota(jnp.int32, 16)
        iota50 = lane16 * HIST
        didx = [lane16 + 16 * c for c in range(DIM // 16)]
        osems = (osem0, osem1)

        def extract(step, gpar):
            btl = step % BT_PER_W
            l = step // BT_PER_W
            base = btl * (128 * HIST) + l
            for k in range(8):
                g = plsc.load_gather(idx_v, [iota50 + (base + k * 16 * HIST)])
                glist_v[gpar, pl.ds(16 * k, 16)] = g

        def start_gather(gpar, slot):
            return pltpu.async_copy(
                table_hbm.at[glist_v.at[gpar]], rows_v.at[slot], gsem
            )

        def wait_gather(slot):
            pltpu.make_async_copy(
                table_hbm.at[pl.ds(0, 128)], rows_v.at[slot], gsem
            ).wait()

        def drain_group(lpar):
            for dt in range(8):
                pltpu.make_async_copy(
                    out_hbm.at[0, dt, pl.ds(0, BT_PER_W)],
                    blk_v.at[lpar, :, pl.ds(dt * 8, 8), pl.ds(0, 128)],
                    osems[lpar],
                ).wait()

        def body(t, l, btl, lpar):
            wait_gather(btl)

            @pl.when(jnp.logical_and(btl == 0, l >= 2))
            def _():
                drain_group(lpar)

            gpar = btl % 2
            extract(jnp.minimum(t + 2, NSTEP - 1), gpar)
            start_gather(gpar, (btl + 2) % BT_PER_W)

            rcur = rows_v.at[btl]
            bcur = blk_v.at[lpar, btl]

            def j_body(j, c2):
                j_bc = lane16 * 0 + j
                for c in range(DIM // 16):
                    v = rcur[j, pl.ds(16 * c, 16)]
                    plsc.store_scatter(bcur, [didx[c], j_bc], v)
                return c2

            lax.fori_loop(0, 128, j_body, 0, unroll=4)

            if btl == BT_PER_W - 1:
                bt0 = wid * BT_PER_W
                for dt in range(8):
                    pltpu.async_copy(
                        blk_v.at[lpar, :, pl.ds(dt * 8, 8), pl.ds(0, 128)],
                        out_hbm.at[l, dt, pl.ds(bt0, BT_PER_W)],
                        osems[lpar],
                    )

        extract(jnp.int32(0), 0)
        start_gather(0, 0)
        extract(jnp.int32(1), 1)
        start_gather(1, 1)

        def l_pair(lp, carry):
            for lpar in range(2):
                l = 2 * lp + lpar
                for btl in range(BT_PER_W):
                    body(l * BT_PER_W + btl, l, btl, lpar)
            return carry

        lax.fori_loop(0, HIST // 2, l_pair, 0)

        wait_gather(0)
        wait_gather(1)
        drain_group(0)
        drain_group(1)

    return emb_kernel


_emb = _build_sc_call()


def kernel(x, table, training):
    del training
    x_flat = x.reshape(-1).astype(jnp.int32)
    out5 = _emb(x_flat, table)
    return out5.transpose(2, 4, 1, 3, 0).reshape(BATCH, DIM, HIST)


# speedup vs baseline: 1.0296x; 1.0296x over previous
import functools

import jax
import jax.numpy as jnp
from jax import lax
from jax.experimental import pallas as pl
from jax.experimental.pallas import tpu as pltpu
from jax.experimental.pallas import tpu_sc as plsc

BATCH = 16384
HIST = 50
DIM = 64

NC = 2
NS = 16
NW = NC * NS

B_PER_W = BATCH // NW
IDX_PER_W = B_PER_W * HIST
BT_PER_W = B_PER_W // 128
NSTEP = BT_PER_W * HIST


def _build_sc_call():
    mesh = plsc.VectorSubcoreMesh(core_axis_name="c", subcore_axis_name="s")

    @functools.partial(
        pl.kernel,
        mesh=mesh,
        compiler_params=pltpu.CompilerParams(
            needs_layout_passes=False, use_tc_tiling_on_sc=False
        ),
        out_type=jax.ShapeDtypeStruct(
            (HIST, DIM // 8, BATCH // 128, 8, 128), jnp.float32
        ),
        scratch_types=[
            pltpu.VMEM((IDX_PER_W,), jnp.int32),
            pltpu.VMEM((128,), jnp.int32),
            pltpu.VMEM((2, 128, DIM), jnp.float32),
            pltpu.VMEM((2, DIM, 129), jnp.float32),
            pltpu.SemaphoreType.DMA,
            pltpu.SemaphoreType.DMA,
            pltpu.SemaphoreType.DMA,
        ],
    )
    def emb_kernel(
        x_hbm, table_hbm, out_hbm, idx_v, glist_v, rows_v, blk_v,
        gsem, osem0, osem1,
    ):
        wid = lax.axis_index("s") * NC + lax.axis_index("c")
        pltpu.sync_copy(x_hbm.at[pl.ds(wid * IDX_PER_W, IDX_PER_W)], idx_v)

        lane16 = lax.iota(jnp.int32, 16)
        iota50 = lane16 * HIST
        didx = [lane16 + 16 * c for c in range(DIM // 16)]
        osems = (osem0, osem1)

        def extract(step):
            btl = step // HIST
            l = step % HIST
            base = btl * (128 * HIST) + l
            for k in range(8):
                g = plsc.load_gather(idx_v, [iota50 + (base + k * 16 * HIST)])
                glist_v[pl.ds(16 * k, 16)] = g

        def start_gather(buf):
            return pltpu.async_copy(
                table_hbm.at[glist_v], rows_v.at[buf], gsem
            )

        def body(t, cur):
            btl = t // HIST
            l = t % HIST
            nxt = 1 - cur

            @pl.when(t >= 2)
            def _():
                for dt in range(8):
                    pltpu.make_async_copy(
                        out_hbm.at[0, 0, 0],
                        blk_v.at[cur, pl.ds(dt * 8, 8), pl.ds(0, 128)],
                        osems[cur],
                    ).wait()

            extract(jnp.minimum(t + 1, NSTEP - 1))
            cp = start_gather(nxt)

            rcur = rows_v.at[cur]
            bcur = blk_v.at[cur]

            def j_body(j, c2):
                j_bc = lane16 * 0 + j
                for c in range(DIM // 16):
                    v = rcur[j, pl.ds(16 * c, 16)]
                    plsc.store_scatter(bcur, [didx[c], j_bc], v)
                return c2

            lax.fori_loop(0, 128, j_body, 0, unroll=4)

            bt = wid * BT_PER_W + btl
            for dt in range(8):
                pltpu.async_copy(
                    blk_v.at[cur, pl.ds(dt * 8, 8), pl.ds(0, 128)],
                    out_hbm.at[l, dt, bt],
                    osems[cur],
                )

            cp.wait()
            return cur

        extract(jnp.int32(0))
        start_gather(0).wait()

        def pair(g, carry):
            body(2 * g, 0)
            body(2 * g + 1, 1)
            return carry

        lax.fori_loop(0, NSTEP // 2, pair, 0)

        for p in range(2):
            for dt in range(8):
                pltpu.make_async_copy(
                    out_hbm.at[0, 0, 0],
                    blk_v.at[p, pl.ds(dt * 8, 8), pl.ds(0, 128)],
                    osems[p],
                ).wait()

    return emb_kernel


_emb = _build_sc_call()


def kernel(x, table, training):
    del training
    x_flat = x.reshape(-1).astype(jnp.int32)
    out5 = _emb(x_flat, table)
    return out5.transpose(2, 4, 1, 3, 0).reshape(BATCH, DIM, HIST)


# speedup vs baseline: 1.0355x; 1.0057x over previous
import functools

import jax
import jax.numpy as jnp
from jax import lax
from jax.experimental import pallas as pl
from jax.experimental.pallas import tpu as pltpu
from jax.experimental.pallas import tpu_sc as plsc

BATCH = 16384
HIST = 50
DIM = 64

NC = 2
NS = 16
NW = NC * NS

B_PER_W = BATCH // NW
IDX_PER_W = B_PER_W * HIST
BT_PER_W = B_PER_W // 128
NSTEP = BT_PER_W * HIST


def _build_sc_call():
    mesh = plsc.VectorSubcoreMesh(core_axis_name="c", subcore_axis_name="s")

    @functools.partial(
        pl.kernel,
        mesh=mesh,
        compiler_params=pltpu.CompilerParams(
            needs_layout_passes=False, use_tc_tiling_on_sc=False
        ),
        out_type=jax.ShapeDtypeStruct(
            (HIST, DIM // 8, BATCH // 128, 8, 128), jnp.float32
        ),
        scratch_types=[
            pltpu.VMEM((IDX_PER_W,), jnp.int32),
            pltpu.VMEM((128,), jnp.int32),
            pltpu.VMEM((2, 128, DIM), jnp.float32),
            pltpu.VMEM((2, DIM, 129), jnp.float32),
            pltpu.SemaphoreType.DMA,
            pltpu.SemaphoreType.DMA,
            pltpu.SemaphoreType.DMA,
        ],
    )
    def emb_kernel(
        x_hbm, table_hbm, out_hbm, idx_v, glist_v, rows_v, blk_v,
        gsem, osem0, osem1,
    ):
        wid = lax.axis_index("s") * NC + lax.axis_index("c")
        pltpu.sync_copy(x_hbm.at[pl.ds(wid * IDX_PER_W, IDX_PER_W)], idx_v)

        lane16 = lax.iota(jnp.int32, 16)
        iota50 = lane16 * HIST
        didx = [lane16 + 16 * c for c in range(DIM // 16)]
        osems = (osem0, osem1)

        def extract(step):
            btl = step // HIST
            l = step % HIST
            base = btl * (128 * HIST) + l
            for k in range(8):
                g = plsc.load_gather(idx_v, [iota50 + (base + k * 16 * HIST)])
                glist_v[pl.ds(16 * k, 16)] = g

        def start_gather(buf):
            return pltpu.async_copy(
                table_hbm.at[glist_v], rows_v.at[buf], gsem
            )

        def body(t, cur):
            btl = t // HIST
            l = t % HIST
            nxt = 1 - cur

            extract(jnp.minimum(t + 1, NSTEP - 1))
            cp = start_gather(nxt)

            @pl.when(t >= 2)
            def _():
                for dt in range(8):
                    pltpu.make_async_copy(
                        out_hbm.at[0, 0, 0],
                        blk_v.at[cur, pl.ds(dt * 8, 8), pl.ds(0, 128)],
                        osems[cur],
                    ).wait()

            rcur = rows_v.at[cur]
            bcur = blk_v.at[cur]

            def j_body(j, c2):
                j_bc = lane16 * 0 + j
                for c in range(DIM // 16):
                    v = rcur[j, pl.ds(16 * c, 16)]
                    plsc.store_scatter(bcur, [didx[c], j_bc], v)
                return c2

            lax.fori_loop(0, 128, j_body, 0, unroll=8)

            bt = wid * BT_PER_W + btl
            for dt in range(8):
                pltpu.async_copy(
                    blk_v.at[cur, pl.ds(dt * 8, 8), pl.ds(0, 128)],
                    out_hbm.at[l, dt, bt],
                    osems[cur],
                )

            cp.wait()
            return cur

        extract(jnp.int32(0))
        start_gather(0).wait()

        def pair(g, carry):
            body(2 * g, 0)
            body(2 * g + 1, 1)
            return carry

        lax.fori_loop(0, NSTEP // 2, pair, 0)

        for p in range(2):
            for dt in range(8):
                pltpu.make_async_copy(
                    out_hbm.at[0, 0, 0],
                    blk_v.at[p, pl.ds(dt * 8, 8), pl.ds(0, 128)],
                    osems[p],
                ).wait()

    return emb_kernel


_emb = _build_sc_call()


def kernel(x, table, training):
    del training
    x_flat = x.reshape(-1).astype(jnp.int32)
    out5 = _emb(x_flat, table)
    return out5.transpose(2, 4, 1, 3, 0).reshape(BATCH, DIM, HIST)
